# baseline (speedup 1.0000x reference)
import functools

import jax
import jax.numpy as jnp
from jax import lax
from jax.experimental import pallas as pl
from jax.experimental.pallas import tpu as pltpu
from jax.experimental.pallas import tpu_sc as plsc

N = 10000
E = 320000
NC = 2
NS = 16
NW = NC * NS
EPW = E // NW
CHUNK = 40
NCHUNK = EPW // CHUNK
RPT = N // NS
NBUF = 5
GA = 3
RB = 1000

_F32 = jnp.float32


def _fill2d(ref, rows, cols, value):
    v = jnp.full((16,), value, _F32)
    nb = cols // 16

    def body(r, _):
        for k in range(nb):
            ref[r, pl.ds(16 * k, 16)] = v
        return 0

    lax.fori_loop(0, rows, body, 0)


def _zero_acc(zb, acc, s):
    _fill2d(zb, ZBR, zb.shape[1], 0.0)

    def body(k, _):
        pltpu.sync_copy(zb, acc.at[pl.ds(s * RPT + k * ZBR, ZBR)])
        return 0

    lax.fori_loop(0, RPT // ZBR, body, 0)


ZBR = 25


def _make_gspmm(dw, packed_out):
    mesh = plsc.VectorSubcoreMesh(core_axis_name="c", subcore_axis_name="s")
    out_shape = (N, 128) if packed_out else (NC, N, 128)
    scratch = [
        pltpu.VMEM((NCHUNK, CHUNK), jnp.int32),
        pltpu.VMEM((NCHUNK, CHUNK), jnp.int32),
        pltpu.VMEM((ZBR, dw), _F32),
        pltpu.VMEM_SHARED((N, dw), _F32),
    ]
    scratch += [pltpu.VMEM((CHUNK, dw), _F32) for _ in range(NBUF)]
    scratch += [pltpu.SemaphoreType.DMA for _ in range(2 * NBUF)]

    @functools.partial(
        pl.kernel, out_type=jax.ShapeDtypeStruct(out_shape, _F32),
        mesh=mesh, scratch_types=tuple(scratch),
        compiler_params=pltpu.CompilerParams(use_tc_tiling_on_sc=False))
    def body(g_hbm, ei_hbm, y_out, src_v, dst_v, zb, acc, *bufsem):
        bufs = bufsem[:NBUF]
        gsem = bufsem[NBUF:2 * NBUF]
        ssem = bufsem[2 * NBUF:]
        c = lax.axis_index("c")
        s = lax.axis_index("s")
        w = s * NC + c

        pltpu.sync_copy(ei_hbm.at[0, w], src_v)
        pltpu.sync_copy(ei_hbm.at[1, w], dst_v)
        _zero_acc(zb, acc, s)
        plsc.subcore_barrier()

        def fire_g(j, b):
            pltpu.async_copy(g_hbm.at[src_v.at[j]], bufs[b], gsem[b])

        def wait_g(j, b):
            pltpu.make_async_copy(
                g_hbm.at[src_v.at[j]], bufs[b], gsem[b]).wait()

        def fire_s(j, b):
            pltpu.async_copy(bufs[b], acc.at[dst_v.at[j]], ssem[b],
                             add=True)

        def wait_s(j, b):
            pltpu.make_async_copy(
                bufs[b], acc.at[dst_v.at[j]], ssem[b]).wait()

        for b in range(GA):
            fire_g(b, b)
        for k in range(NBUF):
            wait_g(k, k)
            fire_s(k, k)
            if k >= 2:
                wait_s(k - 2, k - 2)
            fire_g(k + GA, (k + GA) % NBUF)

        def group(gi, _):
            j0 = gi * NBUF
            for k in range(NBUF):
                j = j0 + k
                wait_g(j, k)
                fire_s(j, k)
                wait_s(j - 2, (k - 2) % NBUF)
                fire_g(j + GA, (k + GA) % NBUF)
            return 0

        lax.fori_loop(1, NCHUNK // NBUF - 1, group, 0)

        j0 = NCHUNK - NBUF
        for k in range(NBUF):
            j = j0 + k
            wait_g(j, k)
            fire_s(j, k)
            wait_s(j - 2, (k - 2) % NBUF)
            if k + GA < NBUF:
                fire_g(j + GA, (k + GA) % NBUF)
        wait_s(NCHUNK - 2, (NBUF - 2) % NBUF)
        wait_s(NCHUNK - 1, NBUF - 1)

        plsc.subcore_barrier()
        if packed_out:
            pltpu.sync_copy(acc.at[pl.ds(s * RPT, RPT)],
                            y_out.at[pl.ds(s * RPT, RPT),
                                     pl.ds(64 * c, 64)])
        elif dw == 128:
            pltpu.sync_copy(acc.at[pl.ds(s * RPT, RPT)],
                            y_out.at[c, pl.ds(s * RPT, RPT)])
        else:
            pltpu.sync_copy(acc.at[pl.ds(s * RPT, RPT)],
                            y_out.at[c, pl.ds(s * RPT, RPT), pl.ds(0, dw)])

    return body


def _make_deg():
    mesh = plsc.VectorSubcoreMesh(core_axis_name="c", subcore_axis_name="s")
    scratch = [
        pltpu.VMEM((NCHUNK, CHUNK), jnp.int32),
        pltpu.VMEM((NCHUNK, CHUNK), jnp.int32),
        pltpu.VMEM((CHUNK, 16), _F32),
        pltpu.VMEM((ZBR, 16), _F32),
        pltpu.VMEM_SHARED((N, 16), _F32),
        pltpu.VMEM_SHARED((N, 16), _F32),
        pltpu.SemaphoreType.DMA,
        pltpu.SemaphoreType.DMA,
    ]

    @functools.partial(
        pl.kernel, out_type=jax.ShapeDtypeStruct((N, 128), _F32),
        mesh=mesh, scratch_types=tuple(scratch),
        compiler_params=pltpu.CompilerParams(use_tc_tiling_on_sc=False))
    def body(ei_hbm, deg_hbm, src_v, dst_v, ones_v, zb, din, dout,
             sem1, sem2):
        c = lax.axis_index("c")
        s = lax.axis_index("s")
        w = s * NC + c

        pltpu.sync_copy(ei_hbm.at[0, w], src_v)
        pltpu.sync_copy(ei_hbm.at[1, w], dst_v)
        _fill2d(ones_v, CHUNK, 16, 1.0)
        _zero_acc(zb, din, s)
        _zero_acc(zb, dout, s)
        plsc.subcore_barrier()

        def fire(j):
            pltpu.async_copy(ones_v, din.at[dst_v.at[j]], sem1, add=True)
            pltpu.async_copy(ones_v, dout.at[src_v.at[j]], sem2, add=True)

        def drain(j):
            pltpu.make_async_copy(ones_v, din.at[dst_v.at[j]],
                                  sem1).wait()
            pltpu.make_async_copy(ones_v, dout.at[src_v.at[j]],
                                  sem2).wait()

        fire(0)

        def chunk(j, _):
            fire(j)
            drain(j - 1)
            return 0

        lax.fori_loop(1, NCHUNK, chunk, 0)
        drain(NCHUNK - 1)
        plsc.subcore_barrier()
        pltpu.sync_copy(din.at[pl.ds(s * RPT, RPT)],
                        deg_hbm.at[pl.ds(s * RPT, RPT),
                                   pl.ds(32 * c, 16)])
        pltpu.sync_copy(dout.at[pl.ds(s * RPT, RPT)],
                        deg_hbm.at[pl.ds(s * RPT, RPT),
                                   pl.ds(32 * c + 16, 16)])

    return body


_deg_kernel = _make_deg()
_spmm128 = _make_gspmm(128, False)
_spmm80 = _make_gspmm(80, False)
_spmm64 = _make_gspmm(64, True)


def _leaky(v):
    return jnp.where(v >= 0, v, 0.01 * v)


def _t1_body(yp, degs, W1, b1, Wg, bg, feat_o, rsqout16_o, invdeg_o,
             rsqin_o):
    d = degs[...]
    deg_in = jnp.maximum(d[:, 0:1] + d[:, 32:33], 1.0)
    deg_out = jnp.maximum(d[:, 16:17] + d[:, 48:49], 1.0)
    invdeg = 1.0 / deg_in
    rsq_in = jnp.sqrt(invdeg)
    rsq_out = lax.rsqrt(deg_out)
    m1 = (yp[0] + yp[1]) * invdeg
    h1 = _leaky(jnp.dot(m1, W1[...],
                        preferred_element_type=_F32) + b1[...])
    feat_o[...] = jnp.dot(h1, Wg[...],
                          preferred_element_type=_F32) * rsq_out
    rsqout16_o[...] = jnp.broadcast_to(rsq_out, (RB, 16))
    invdeg_o[...] = invdeg
    rsqin_o[...] = rsq_in


_t1 = pl.pallas_call(
    _t1_body,
    grid=(N // RB,),
    in_specs=[
        pl.BlockSpec((NC, RB, 128), lambda i: (0, i, 0)),
        pl.BlockSpec((RB, 128), lambda i: (i, 0)),
        pl.BlockSpec((128, 128), lambda i: (0, 0)),
        pl.BlockSpec((128,), lambda i: (0,)),
        pl.BlockSpec((128, 128), lambda i: (0, 0)),
        pl.BlockSpec((128,), lambda i: (0,)),
    ],
    out_specs=(pl.BlockSpec((RB, 128), lambda i: (i, 0)),
               pl.BlockSpec((RB, 16), lambda i: (i, 0)),
               pl.BlockSpec((RB, 1), lambda i: (i, 0)),
               pl.BlockSpec((RB, 1), lambda i: (i, 0))),
    out_shape=(jax.ShapeDtypeStruct((N, 128), _F32),
               jax.ShapeDtypeStruct((N, 16), _F32),
               jax.ShapeDtypeStruct((N, 1), _F32),
               jax.ShapeDtypeStruct((N, 1), _F32)),
)


def _t2_body(yp, rsqin, rsqout16, bg, W2, g3_o):
    agg2 = (yp[0] + yp[1]) * rsqin[...] + bg[...]
    h2 = _leaky(agg2)
    g3 = jnp.dot(h2, W2[...], preferred_element_type=_F32)
    g3_o[...] = jnp.concatenate([g3, rsqout16[...]], axis=1)


_t2 = pl.pallas_call(
    _t2_body,
    grid=(N // RB,),
    in_specs=[
        pl.BlockSpec((NC, RB, 128), lambda i: (0, i, 0)),
        pl.BlockSpec((RB, 1), lambda i: (i, 0)),
        pl.BlockSpec((RB, 16), lambda i: (i, 0)),
        pl.BlockSpec((128,), lambda i: (0,)),
        pl.BlockSpec((128, 64), lambda i: (0, 0)),
    ],
    out_specs=pl.BlockSpec((RB, 80), lambda i: (i, 0)),
    out_shape=jax.ShapeDtypeStruct((N, 80), _F32),
)


def _t3_body(yp, invdeg, rsqin, b2, Wg2, bg2, W3, g5_o):
    h3 = (yp[0, :, :64] + yp[1, :, :64]) * invdeg[...] + b2[...]
    pooled = jnp.mean(h3, axis=0, keepdims=True)
    q = jnp.dot(pooled, Wg2[...], preferred_element_type=_F32)
    alpha = (yp[0, :, 64:65] + yp[1, :, 64:65]) * rsqin[...]
    h4 = _leaky(alpha * q + bg2[...])
    g5_o[...] = jnp.dot(h4, W3[...], preferred_element_type=_F32)


_t3 = pl.pallas_call(
    _t3_body,
    out_shape=jax.ShapeDtypeStruct((N, 64), _F32),
)


def _t4_body(yp, invdeg, b3, out_o):
    y = yp[...]
    out_o[...] = (y[:, :64] + y[:, 64:]) * invdeg[...] + b3[...]


_t4 = pl.pallas_call(
    _t4_body,
    grid=(N // RB,),
    in_specs=[
        pl.BlockSpec((RB, 128), lambda i: (i, 0)),
        pl.BlockSpec((RB, 1), lambda i: (i, 0)),
        pl.BlockSpec((64,), lambda i: (0,)),
    ],
    out_specs=pl.BlockSpec((RB, 64), lambda i: (i, 0)),
    out_shape=jax.ShapeDtypeStruct((N, 64), _F32),
)


def kernel(x, edge_index, W1, b1, Wg, bg, W2, b2, Wg2, bg2, W3, b3):
    ei_r = edge_index.reshape(2, NW, NCHUNK, CHUNK)

    degs = _deg_kernel(ei_r)
    y1 = _spmm128(x, ei_r)
    feat2, rsqout16, invdeg, rsqin = _t1(y1, degs, W1, b1, Wg, bg)
    y2 = _spmm128(feat2, ei_r)
    g3ext = _t2(y2, rsqin, rsqout16, bg, W2)
    y3 = _spmm80(g3ext, ei_r)
    g5 = _t3(y3, invdeg, rsqin, b2, Wg2, bg2, W3)
    y5 = _spmm64(g5, ei_r)
    out = _t4(y5, invdeg, b3)
    return out


# speedup vs baseline: 1.1943x; 1.1943x over previous
import functools

import jax
import jax.numpy as jnp
from jax import lax
from jax.experimental import pallas as pl
from jax.experimental.pallas import tpu as pltpu
from jax.experimental.pallas import tpu_sc as plsc

N = 10000
E = 320000
NC = 2
NS = 16
NW = NC * NS
EPW = E // NW
RPT = N // NS
NBUF = 5
ZBR = 25

_F32 = jnp.float32


def _fill2d(ref, rows, cols, value):
    v = jnp.full((16,), value, _F32)
    nb = cols // 16

    def body(r, _):
        for k in range(nb):
            ref[r, pl.ds(16 * k, 16)] = v
        return 0

    lax.fori_loop(0, rows, body, 0)


def _zero_acc(zb, acc, s):
    _fill2d(zb, ZBR, zb.shape[1], 0.0)

    def body(k, _):
        pltpu.sync_copy(zb, acc.at[pl.ds(s * RPT + k * ZBR, ZBR)])
        return 0

    lax.fori_loop(0, RPT // ZBR, body, 0)


def _make_gspmm(dw, chunk):
    nchunk = EPW // chunk
    assert nchunk % NBUF == 0
    mesh = plsc.VectorSubcoreMesh(core_axis_name="c", subcore_axis_name="s")
    scratch = [
        pltpu.VMEM((nchunk, chunk), jnp.int32),
        pltpu.VMEM((nchunk, chunk), jnp.int32),
        pltpu.VMEM((ZBR, dw), _F32),
        pltpu.VMEM_SHARED((N, dw), _F32),
    ]
    scratch += [pltpu.VMEM((chunk, dw), _F32) for _ in range(NBUF)]
    scratch += [pltpu.SemaphoreType.DMA for _ in range(NBUF)]

    @functools.partial(
        pl.kernel, out_type=jax.ShapeDtypeStruct((NC, N, 128), _F32),
        mesh=mesh, scratch_types=tuple(scratch),
        compiler_params=pltpu.CompilerParams(use_tc_tiling_on_sc=False))
    def body(g_hbm, ei_hbm, y_out, src_v, dst_v, zb, acc, *bufsem):
        bufs = bufsem[:NBUF]
        sems = bufsem[NBUF:]
        c = lax.axis_index("c")
        s = lax.axis_index("s")
        w = s * NC + c

        pltpu.sync_copy(ei_hbm.at[0, w], src_v)
        pltpu.sync_copy(ei_hbm.at[1, w], dst_v)
        _zero_acc(zb, acc, s)
        plsc.subcore_barrier()

        def fire(j, b):
            pltpu.async_copy(g_hbm.at[src_v.at[j]], bufs[b], sems[b])

        def drain_and_scatter(j, b):
            pltpu.make_async_copy(
                g_hbm.at[src_v.at[j]], bufs[b], sems[b]).wait()
            pltpu.sync_copy(bufs[b], acc.at[dst_v.at[j]], add=True)

        for b in range(NBUF):
            fire(b, b)

        def group(gi, _):
            j0 = gi * NBUF
            for b in range(NBUF):
                drain_and_scatter(j0 + b, b)
                fire(j0 + b + NBUF, b)
            return 0

        lax.fori_loop(0, nchunk // NBUF - 1, group, 0)
        for b in range(NBUF):
            drain_and_scatter(nchunk - NBUF + b, b)

        plsc.subcore_barrier()
        if dw == 128:
            pltpu.sync_copy(acc.at[pl.ds(s * RPT, RPT)],
                            y_out.at[c, pl.ds(s * RPT, RPT)])
        else:
            pltpu.sync_copy(acc.at[pl.ds(s * RPT, RPT)],
                            y_out.at[c, pl.ds(s * RPT, RPT), pl.ds(0, dw)])

    return body


def _make_deg(chunk):
    nchunk = EPW // chunk
    mesh = plsc.VectorSubcoreMesh(core_axis_name="c", subcore_axis_name="s")
    scratch = [
        pltpu.VMEM((nchunk, chunk), jnp.int32),
        pltpu.VMEM((nchunk, chunk), jnp.int32),
        pltpu.VMEM((chunk, 16), _F32),
        pltpu.VMEM((ZBR, 16), _F32),
        pltpu.VMEM_SHARED((N, 16), _F32),
        pltpu.VMEM_SHARED((N, 16), _F32),
        pltpu.SemaphoreType.DMA,
        pltpu.SemaphoreType.DMA,
    ]

    @functools.partial(
        pl.kernel, out_type=jax.ShapeDtypeStruct((NC, N, 128), _F32),
        mesh=mesh, scratch_types=tuple(scratch),
        compiler_params=pltpu.CompilerParams(use_tc_tiling_on_sc=False))
    def body(ei_hbm, deg_hbm, src_v, dst_v, ones_v, zb, din, dout,
             sem1, sem2):
        c = lax.axis_index("c")
        s = lax.axis_index("s")
        w = s * NC + c

        pltpu.sync_copy(ei_hbm.at[0, w], src_v)
        pltpu.sync_copy(ei_hbm.at[1, w], dst_v)
        _fill2d(ones_v, chunk, 16, 1.0)
        _zero_acc(zb, din, s)
        _zero_acc(zb, dout, s)
        plsc.subcore_barrier()

        def fire(j):
            pltpu.async_copy(ones_v, din.at[dst_v.at[j]], sem1, add=True)
            pltpu.async_copy(ones_v, dout.at[src_v.at[j]], sem2, add=True)

        def drain(j):
            pltpu.make_async_copy(ones_v, din.at[dst_v.at[j]],
                                  sem1).wait()
            pltpu.make_async_copy(ones_v, dout.at[src_v.at[j]],
                                  sem2).wait()

        fire(0)

        def chunk_body(j, _):
            fire(j)
            drain(j - 1)
            return 0

        lax.fori_loop(1, nchunk, chunk_body, 0)
        drain(nchunk - 1)
        plsc.subcore_barrier()
        pltpu.sync_copy(din.at[pl.ds(s * RPT, RPT)],
                        deg_hbm.at[c, pl.ds(s * RPT, RPT), pl.ds(0, 16)])
        pltpu.sync_copy(dout.at[pl.ds(s * RPT, RPT)],
                        deg_hbm.at[c, pl.ds(s * RPT, RPT), pl.ds(16, 16)])

    return body


CH40, CH80 = 40, 80
_deg_kernel = _make_deg(CH80)
_spmm128 = _make_gspmm(128, CH40)
_spmm80 = _make_gspmm(80, CH80)
_spmm64 = _make_gspmm(64, CH80)


def _leaky(v):
    return jnp.where(v >= 0, v, 0.01 * v)


def _t1_body(yp, degs, W1, b1, Wg, bg, feat_o, rsqout16_o, invdeg_o,
             rsqin_o):
    deg_in = jnp.maximum(degs[0, :, 0:1] + degs[1, :, 0:1], 1.0)
    deg_out = jnp.maximum(degs[0, :, 16:17] + degs[1, :, 16:17], 1.0)
    invdeg = 1.0 / deg_in
    rsq_in = jnp.sqrt(invdeg)
    rsq_out = lax.rsqrt(deg_out)
    m1 = (yp[0] + yp[1]) * invdeg
    h1 = _leaky(jnp.dot(m1, W1[...],
                        preferred_element_type=_F32) + b1[...])
    feat_o[...] = jnp.dot(h1, Wg[...],
                          preferred_element_type=_F32) * rsq_out
    rsqout16_o[...] = jnp.broadcast_to(rsq_out, (N, 16))
    invdeg_o[...] = invdeg
    rsqin_o[...] = rsq_in


_t1 = pl.pallas_call(
    _t1_body,
    out_shape=(jax.ShapeDtypeStruct((N, 128), _F32),
               jax.ShapeDtypeStruct((N, 16), _F32),
               jax.ShapeDtypeStruct((N, 1), _F32),
               jax.ShapeDtypeStruct((N, 1), _F32)),
)


def _t2_body(yp, rsqin, rsqout16, bg, W2, g3_o):
    agg2 = (yp[0] + yp[1]) * rsqin[...] + bg[...]
    h2 = _leaky(agg2)
    g3 = jnp.dot(h2, W2[...], preferred_element_type=_F32)
    g3_o[...] = jnp.concatenate([g3, rsqout16[...]], axis=1)


_t2 = pl.pallas_call(
    _t2_body,
    out_shape=jax.ShapeDtypeStruct((N, 80), _F32),
)


def _t3_body(yp, invdeg, rsqin, b2, Wg2, bg2, W3, g5_o):
    h3 = (yp[0, :, :64] + yp[1, :, :64]) * invdeg[...] + b2[...]
    pooled = jnp.mean(h3, axis=0, keepdims=True)
    q = jnp.dot(pooled, Wg2[...], preferred_element_type=_F32)
    alpha = (yp[0, :, 64:65] + yp[1, :, 64:65]) * rsqin[...]
    h4 = _leaky(alpha * q + bg2[...])
    g5_o[...] = jnp.dot(h4, W3[...], preferred_element_type=_F32)


_t3 = pl.pallas_call(
    _t3_body,
    out_shape=jax.ShapeDtypeStruct((N, 64), _F32),
)


def _t4_body(yp, invdeg, b3, out_o):
    out_o[...] = (yp[0, :, :64] + yp[1, :, :64]) * invdeg[...] + b3[...]


_t4 = pl.pallas_call(
    _t4_body,
    out_shape=jax.ShapeDtypeStruct((N, 64), _F32),
)


def kernel(x, edge_index, W1, b1, Wg, bg, W2, b2, Wg2, bg2, W3, b3):
    ei40 = edge_index.reshape(2, NW, EPW // CH40, CH40)
    ei80 = edge_index.reshape(2, NW, EPW // CH80, CH80)

    degs = _deg_kernel(ei80)
    y1 = _spmm128(x, ei40)
    feat2, rsqout16, invdeg, rsqin = _t1(y1, degs, W1, b1, Wg, bg)
    y2 = _spmm128(feat2, ei40)
    g3ext = _t2(y2, rsqin, rsqout16, bg, W2)
    y3 = _spmm80(g3ext, ei80)
    g5 = _t3(y3, invdeg, rsqin, b2, Wg2, bg2, W3)
    y5 = _spmm64(g5, ei80)
    out = _t4(y5, invdeg, b3)
    return out
